# baseline (speedup 1.0000x reference)
import functools

import jax
import jax.numpy as jnp
from jax import lax
from jax.experimental import pallas as pl
from jax.experimental.pallas import tpu as pltpu
from jax.experimental.pallas import tpu_sc as plsc

EMB = 128
L = 16
BLK = 8192
NW = 32
SC_ROWS_W = 3200
SC_CHUNK = 80


def _tc_body(flag_ref, w_ref, b_ref, out_ref):
    w = w_ref[...]
    b = b_ref[...]
    dn = (((0,), (0,)), ((), ()))
    fl = flag_ref[...].reshape(1, BLK)
    out_ref[...] = lax.dot_general(
        fl, w, dn, preferred_element_type=jnp.float32) + b


def _tc_fill(flag, w_row, b_row, n):
    return pl.pallas_call(
        _tc_body,
        grid=(pl.cdiv(n, BLK),),
        in_specs=[
            pl.BlockSpec((BLK,), lambda i: (i,)),
            pl.BlockSpec((1, EMB), lambda i: (0, 0)),
            pl.BlockSpec((1, EMB), lambda i: (0, 0)),
        ],
        out_specs=pl.BlockSpec((BLK, EMB), lambda i: (i, 0)),
        out_shape=jax.ShapeDtypeStruct((n, EMB), jnp.float32),
    )(flag, w_row, b_row)


def _sc_fill(n):
    mesh = plsc.VectorSubcoreMesh(core_axis_name="c", subcore_axis_name="s")

    @functools.partial(
        pl.kernel,
        mesh=mesh,
        out_type=jax.ShapeDtypeStruct((n, EMB), jnp.float32),
        scratch_types=[
            pltpu.VMEM((EMB,), jnp.float32),
            pltpu.VMEM((EMB,), jnp.float32),
            pltpu.VMEM((SC_ROWS_W,), jnp.float32),
            pltpu.VMEM((SC_CHUNK, EMB), jnp.float32),
        ],
    )
    def sc_kernel(flag_hbm, w_hbm, b_hbm, out_hbm, w_v, b_v, flags_v, buf):
        wid = lax.axis_index("s") * 2 + lax.axis_index("c")
        row0 = wid * SC_ROWS_W
        pltpu.sync_copy(w_hbm, w_v)
        pltpu.sync_copy(b_hbm, b_v)
        pltpu.sync_copy(flag_hbm.at[pl.ds(row0, SC_ROWS_W)], flags_v)
        wregs = [w_v[pl.ds(j * L, L)] for j in range(8)]
        bregs = [b_v[pl.ds(j * L, L)] for j in range(8)]
        rows_here = jnp.minimum(SC_ROWS_W, n - row0)
        n_chunks = rows_here // SC_CHUNK

        def chunk_body(c, carry):
            base = c * SC_CHUNK
            gbase = row0 + base

            def grp_body(gi, cc):
                fl16 = flags_v[pl.ds(base + gi * L, L)]
                for k in range(L):
                    f = fl16[k]
                    r = gi * L + k
                    for j in range(8):
                        buf[r, pl.ds(j * L, L)] = f * wregs[j] + bregs[j]
                return cc

            lax.fori_loop(0, SC_CHUNK // L, grp_body, 0)
            pltpu.sync_copy(buf, out_hbm.at[pl.ds(gbase, SC_CHUNK)])
            return carry

        lax.fori_loop(0, n_chunks, chunk_body, 0)

    return sc_kernel


def kernel(x_lig, x_rec, v_lig, v_rec, aa_rec, batch_idx_lig, batch_idx_rec,
           lig_flag, rec_flag, W_ind, b_ind):
    n = lig_flag.shape[0]
    w_row = W_ind.reshape(1, EMB)
    b_row = b_ind.reshape(1, EMB)
    h_lig = _tc_fill(lig_flag, w_row, b_row, n)
    rec_pad = jnp.pad(rec_flag, (0, NW * SC_ROWS_W - n))
    h_rec = _sc_fill(n)(rec_pad, W_ind.reshape(EMB), b_ind)
    return (x_lig, x_rec, h_lig, h_rec)


# speedup vs baseline: 1.0013x; 1.0013x over previous
import functools

import jax
import jax.numpy as jnp
from jax import lax
from jax.experimental import pallas as pl
from jax.experimental.pallas import tpu as pltpu
from jax.experimental.pallas import tpu_sc as plsc

EMB = 128
L = 16
BLK = 8192
NW = 32
SC_ROWS_W = 3200
SC_CHUNK = 80


def _tc_body(flag_ref, w_ref, b_ref, out_ref):
    w = w_ref[...]
    b = b_ref[...]
    dn = (((0,), (0,)), ((), ()))
    fl = flag_ref[...].reshape(1, BLK)
    out_ref[...] = lax.dot_general(
        fl, w, dn, preferred_element_type=jnp.float32) + b


def _tc_fill(flag, w_row, b_row, n):
    return pl.pallas_call(
        _tc_body,
        grid=(pl.cdiv(n, BLK),),
        in_specs=[
            pl.BlockSpec((BLK,), lambda i: (i,)),
            pl.BlockSpec((1, EMB), lambda i: (0, 0)),
            pl.BlockSpec((1, EMB), lambda i: (0, 0)),
        ],
        out_specs=pl.BlockSpec((BLK, EMB), lambda i: (i, 0)),
        out_shape=jax.ShapeDtypeStruct((n, EMB), jnp.float32),
    )(flag, w_row, b_row)


def _sc_fill(n):
    mesh = plsc.VectorSubcoreMesh(core_axis_name="c", subcore_axis_name="s")

    @functools.partial(
        pl.kernel,
        mesh=mesh,
        out_type=jax.ShapeDtypeStruct((n, EMB), jnp.float32),
        scratch_types=[
            pltpu.VMEM((EMB,), jnp.float32),
            pltpu.VMEM((EMB,), jnp.float32),
            pltpu.VMEM((SC_ROWS_W,), jnp.float32),
            pltpu.VMEM((SC_CHUNK, EMB), jnp.float32),
            pltpu.VMEM((SC_CHUNK, EMB), jnp.float32),
            pltpu.SemaphoreType.DMA,
            pltpu.SemaphoreType.DMA,
        ],
    )
    def sc_kernel(flag_hbm, w_hbm, b_hbm, out_hbm,
                  w_v, b_v, flags_v, buf0, buf1, sem0, sem1):
        wid = lax.axis_index("s") * 2 + lax.axis_index("c")
        row0 = wid * SC_ROWS_W
        pltpu.sync_copy(w_hbm, w_v)
        pltpu.sync_copy(b_hbm, b_v)
        pltpu.sync_copy(flag_hbm.at[pl.ds(row0, SC_ROWS_W)], flags_v)
        wregs = [w_v[pl.ds(j * L, L)] for j in range(8)]
        bregs = [b_v[pl.ds(j * L, L)] for j in range(8)]
        rows_here = jnp.minimum(SC_ROWS_W, n - row0)
        n_pairs = rows_here // (2 * SC_CHUNK)

        def fill_chunk(base, buf):
            def grp_body(gi, cc):
                fl16 = flags_v[pl.ds(base + gi * L, L)]
                for k in range(L):
                    f = fl16[k]
                    r = gi * L + k
                    for j in range(8):
                        buf[r, pl.ds(j * L, L)] = f * wregs[j] + bregs[j]
                return cc

            lax.fori_loop(0, SC_CHUNK // L, grp_body, 0)

        def pair_body(g, carry):
            base0 = g * 2 * SC_CHUNK
            base1 = base0 + SC_CHUNK
            fill_chunk(base0, buf0)
            h0 = pltpu.async_copy(
                buf0, out_hbm.at[pl.ds(row0 + base0, SC_CHUNK)], sem0)
            fill_chunk(base1, buf1)
            h1 = pltpu.async_copy(
                buf1, out_hbm.at[pl.ds(row0 + base1, SC_CHUNK)], sem1)
            h0.wait()
            h1.wait()
            return carry

        lax.fori_loop(0, n_pairs, pair_body, 0)

    return sc_kernel


def kernel(x_lig, x_rec, v_lig, v_rec, aa_rec, batch_idx_lig, batch_idx_rec,
           lig_flag, rec_flag, W_ind, b_ind):
    n = lig_flag.shape[0]
    w_row = W_ind.reshape(1, EMB)
    b_row = b_ind.reshape(1, EMB)
    rec_pad = jnp.pad(rec_flag, (0, NW * SC_ROWS_W - n))
    h_rec = _sc_fill(n)(rec_pad, W_ind.reshape(EMB), b_ind)
    h_lig = _tc_fill(lig_flag, w_row, b_row, n)
    return (x_lig, x_rec, h_lig, h_rec)


# speedup vs baseline: 1.0069x; 1.0056x over previous
import functools

import jax
import jax.numpy as jnp
from jax import lax
from jax.experimental import pallas as pl
from jax.experimental.pallas import tpu as pltpu
from jax.experimental.pallas import tpu_sc as plsc

EMB = 128
L = 16
BLK = 8192
NW = 32
SC_ROWS_W = 3200
SC_CHUNK = 80


def _tc_body(flag_ref, w_ref, b_ref, out_ref):
    w = w_ref[...]
    b = b_ref[...]
    dn = (((0,), (0,)), ((), ()))
    fl = flag_ref[...].reshape(1, BLK)
    out_ref[...] = lax.dot_general(
        fl, w, dn, preferred_element_type=jnp.float32) + b


def _tc_fill(flag, w_row, b_row, n):
    return pl.pallas_call(
        _tc_body,
        grid=(pl.cdiv(n, BLK),),
        in_specs=[
            pl.BlockSpec((BLK,), lambda i: (i,)),
            pl.BlockSpec((1, EMB), lambda i: (0, 0)),
            pl.BlockSpec((1, EMB), lambda i: (0, 0)),
        ],
        out_specs=pl.BlockSpec((BLK, EMB), lambda i: (i, 0)),
        out_shape=jax.ShapeDtypeStruct((n, EMB), jnp.float32),
    )(flag, w_row, b_row)


def _sc_fill(n):
    mesh = plsc.VectorSubcoreMesh(core_axis_name="c", subcore_axis_name="s")

    @functools.partial(
        pl.kernel,
        mesh=mesh,
        out_type=jax.ShapeDtypeStruct((n, EMB), jnp.float32),
        scratch_types=[
            pltpu.VMEM((EMB,), jnp.float32),
            pltpu.VMEM((EMB,), jnp.float32),
            pltpu.VMEM((SC_ROWS_W,), jnp.float32),
            pltpu.VMEM((SC_CHUNK, EMB), jnp.float32),
            pltpu.VMEM((SC_CHUNK, EMB), jnp.float32),
            pltpu.SemaphoreType.DMA,
            pltpu.SemaphoreType.DMA,
        ],
    )
    def sc_kernel(flag_hbm, w_hbm, b_hbm, out_hbm,
                  w_v, b_v, flags_v, buf0, buf1, sem0, sem1):
        wid = lax.axis_index("s") * 2 + lax.axis_index("c")
        row0 = wid * SC_ROWS_W
        pltpu.sync_copy(w_hbm, w_v)
        pltpu.sync_copy(b_hbm, b_v)
        pltpu.sync_copy(flag_hbm.at[pl.ds(row0, SC_ROWS_W)], flags_v)
        wregs = [w_v[pl.ds(j * L, L)] for j in range(8)]
        bregs = [b_v[pl.ds(j * L, L)] for j in range(8)]
        rows_here = jnp.minimum(SC_ROWS_W, n - row0)
        n_pairs = rows_here // (2 * SC_CHUNK)

        def fill_chunk(base, buf):
            fls = [flags_v[pl.ds(base + gi * L, L)] for gi in range(SC_CHUNK // L)]
            for gi in range(SC_CHUNK // L):
                for k in range(L):
                    f = fls[gi][k]
                    r = gi * L + k
                    for j in range(8):
                        buf[r, pl.ds(j * L, L)] = f * wregs[j] + bregs[j]

        def pair_body(g, carry):
            base0 = g * 2 * SC_CHUNK
            base1 = base0 + SC_CHUNK
            fill_chunk(base0, buf0)
            h0 = pltpu.async_copy(
                buf0, out_hbm.at[pl.ds(row0 + base0, SC_CHUNK)], sem0)
            fill_chunk(base1, buf1)
            h1 = pltpu.async_copy(
                buf1, out_hbm.at[pl.ds(row0 + base1, SC_CHUNK)], sem1)
            h0.wait()
            h1.wait()
            return carry

        lax.fori_loop(0, n_pairs, pair_body, 0)

    return sc_kernel


def kernel(x_lig, x_rec, v_lig, v_rec, aa_rec, batch_idx_lig, batch_idx_rec,
           lig_flag, rec_flag, W_ind, b_ind):
    n = lig_flag.shape[0]
    w_row = W_ind.reshape(1, EMB)
    b_row = b_ind.reshape(1, EMB)
    rec_pad = jnp.pad(rec_flag, (0, NW * SC_ROWS_W - n))
    h_rec = _sc_fill(n)(rec_pad, W_ind.reshape(EMB), b_ind)
    h_lig = _tc_fill(lig_flag, w_row, b_row, n)
    return (x_lig, x_rec, h_lig, h_rec)


# speedup vs baseline: 1.0352x; 1.0281x over previous
import functools

import jax
import jax.numpy as jnp
from jax import lax
from jax.experimental import pallas as pl
from jax.experimental.pallas import tpu as pltpu
from jax.experimental.pallas import tpu_sc as plsc

EMB = 128
L = 16
BLK = 8192
NW = 32
SC_ROWS_W = 3200
SC_CHUNK = 400


def _tc_body(flag_ref, w_ref, b_ref, out_ref):
    w = w_ref[...]
    b = b_ref[...]
    dn = (((0,), (0,)), ((), ()))
    fl = flag_ref[...].reshape(1, BLK)
    out_ref[...] = lax.dot_general(
        fl, w, dn, preferred_element_type=jnp.float32) + b


def _tc_fill(flag, w_row, b_row, n):
    return pl.pallas_call(
        _tc_body,
        grid=(pl.cdiv(n, BLK),),
        in_specs=[
            pl.BlockSpec((BLK,), lambda i: (i,)),
            pl.BlockSpec((1, EMB), lambda i: (0, 0)),
            pl.BlockSpec((1, EMB), lambda i: (0, 0)),
        ],
        out_specs=pl.BlockSpec((BLK, EMB), lambda i: (i, 0)),
        out_shape=jax.ShapeDtypeStruct((n, EMB), jnp.float32),
    )(flag, w_row, b_row)


def _sc_fill(n):
    mesh = plsc.VectorSubcoreMesh(core_axis_name="c", subcore_axis_name="s")

    @functools.partial(
        pl.kernel,
        mesh=mesh,
        out_type=jax.ShapeDtypeStruct((n, EMB), jnp.float32),
        scratch_types=[
            pltpu.VMEM((EMB,), jnp.float32),
            pltpu.VMEM((EMB,), jnp.float32),
            pltpu.VMEM((SC_ROWS_W,), jnp.float32),
            pltpu.VMEM((SC_CHUNK, EMB), jnp.float32),
            pltpu.VMEM((SC_CHUNK, EMB), jnp.float32),
            pltpu.SemaphoreType.DMA,
            pltpu.SemaphoreType.DMA,
        ],
    )
    def sc_kernel(flag_hbm, w_hbm, b_hbm, out_hbm,
                  w_v, b_v, flags_v, buf0, buf1, sem0, sem1):
        wid = lax.axis_index("s") * 2 + lax.axis_index("c")
        row0 = wid * SC_ROWS_W
        pltpu.sync_copy(w_hbm, w_v)
        pltpu.sync_copy(b_hbm, b_v)
        pltpu.sync_copy(flag_hbm.at[pl.ds(row0, SC_ROWS_W)], flags_v)
        wregs = [w_v[pl.ds(j * L, L)] for j in range(8)]
        bregs = [b_v[pl.ds(j * L, L)] for j in range(8)]
        rows_here = jnp.minimum(SC_ROWS_W, n - row0)
        n_pairs = rows_here // (2 * SC_CHUNK)

        def fill_chunk(base, buf):
            def grp_body(gi, cc):
                fl16 = flags_v[pl.ds(base + gi * L, L)]
                for k in range(L):
                    f = fl16[k]
                    r = gi * L + k
                    for j in range(8):
                        buf[r, pl.ds(j * L, L)] = f * wregs[j] + bregs[j]
                return cc

            lax.fori_loop(0, SC_CHUNK // L, grp_body, 0)

        def pair_body(g, carry):
            base0 = g * 2 * SC_CHUNK
            base1 = base0 + SC_CHUNK
            fill_chunk(base0, buf0)
            h0 = pltpu.async_copy(
                buf0, out_hbm.at[pl.ds(row0 + base0, SC_CHUNK)], sem0)
            fill_chunk(base1, buf1)
            h1 = pltpu.async_copy(
                buf1, out_hbm.at[pl.ds(row0 + base1, SC_CHUNK)], sem1)
            h0.wait()
            h1.wait()
            return carry

        lax.fori_loop(0, n_pairs, pair_body, 0)

    return sc_kernel


def kernel(x_lig, x_rec, v_lig, v_rec, aa_rec, batch_idx_lig, batch_idx_rec,
           lig_flag, rec_flag, W_ind, b_ind):
    n = lig_flag.shape[0]
    w_row = W_ind.reshape(1, EMB)
    b_row = b_ind.reshape(1, EMB)
    rec_pad = jnp.pad(rec_flag, (0, NW * SC_ROWS_W - n))
    h_rec = _sc_fill(n)(rec_pad, W_ind.reshape(EMB), b_ind)
    h_lig = _tc_fill(lig_flag, w_row, b_row, n)
    return (x_lig, x_rec, h_lig, h_rec)


# speedup vs baseline: 1.5489x; 1.4963x over previous
import jax
import jax.numpy as jnp
from jax.experimental import pallas as pl

EMB = 128
BLK = 16384


def _fill_body(flag_l_ref, flag_r_ref, w_ref, b_ref, out_l_ref, out_r_ref):
    w = w_ref[...]
    b = b_ref[...]
    dn = (((0,), (0,)), ((), ()))
    fl = flag_l_ref[...].reshape(1, BLK)
    fr = flag_r_ref[...].reshape(1, BLK)
    out_l_ref[...] = jax.lax.dot_general(
        fl, w, dn, preferred_element_type=jnp.float32) + b
    out_r_ref[...] = jax.lax.dot_general(
        fr, w, dn, preferred_element_type=jnp.float32) + b


def kernel(x_lig, x_rec, v_lig, v_rec, aa_rec, batch_idx_lig, batch_idx_rec,
           lig_flag, rec_flag, W_ind, b_ind):
    n_lig = lig_flag.shape[0]
    n_rec = rec_flag.shape[0]
    assert n_lig == n_rec
    n = n_lig
    grid = (pl.cdiv(n, BLK),)

    flag_l = lig_flag
    flag_r = rec_flag
    w_row = W_ind.reshape(1, EMB)
    b_row = b_ind.reshape(1, EMB)

    h_lig, h_rec = pl.pallas_call(
        _fill_body,
        grid=grid,
        in_specs=[
            pl.BlockSpec((BLK,), lambda i: (i,)),
            pl.BlockSpec((BLK,), lambda i: (i,)),
            pl.BlockSpec((1, EMB), lambda i: (0, 0)),
            pl.BlockSpec((1, EMB), lambda i: (0, 0)),
        ],
        out_specs=[
            pl.BlockSpec((BLK, EMB), lambda i: (i, 0)),
            pl.BlockSpec((BLK, EMB), lambda i: (i, 0)),
        ],
        out_shape=[
            jax.ShapeDtypeStruct((n, EMB), jnp.float32),
            jax.ShapeDtypeStruct((n, EMB), jnp.float32),
        ],
    )(flag_l, flag_r, w_row, b_row)

    return (x_lig, x_rec, h_lig, h_rec)
